# Initial kernel scaffold; baseline (speedup 1.0000x reference)
import jax
import jax.numpy as jnp
from jax.experimental import pallas as pl


def kernel(_hidden, obs1, obs2, emb_W, emb_b, W_ih, W_hh, b_ih, b_hh, out_W, out_b):
    raise NotImplementedError("write your pallas kernel here")


# speedup vs baseline: 11.4301x; 11.4301x over previous
import jax
import jax.numpy as jnp
from jax.experimental import pallas as pl

_N = 2048
_R = 256
_H = 1024
_OUT = 256
_K = 8


def _body(o1T, o2T, o1b, o2b, embWT, embb, WihT, bsum, oWT, ob, out_ref):
    i = pl.program_id(0)
    pxj = o2T[0:1, :]
    pyj = o2T[1:2, :]
    vxj = pxj - o1T[0:1, :]
    vyj = pyj - o1T[1:2, :]
    pxi = o2b[:, 0:1]
    pyi = o2b[:, 1:2]
    vxi = pxi - o1b[:, 0:1]
    vyi = pyi - o1b[:, 1:2]
    dx = pxj - pxi
    dy = pyj - pyi
    d2 = dx * dx + dy * dy
    colio = jax.lax.broadcasted_iota(jnp.int32, (_R, _N), 1)
    rowg = i * _R + jax.lax.broadcasted_iota(jnp.int32, (_R, _N), 0)
    d2 = jnp.where(colio == rowg, jnp.inf, d2)
    xs = []
    for _ in range(_K):
        m = jnp.min(d2, axis=1, keepdims=True)
        idx = jnp.min(jnp.where(d2 == m, colio, _N), axis=1, keepdims=True)
        sel = colio == idx
        nx = jnp.sum(jnp.where(sel, pxj, 0.0), axis=1, keepdims=True)
        ny = jnp.sum(jnp.where(sel, pyj, 0.0), axis=1, keepdims=True)
        nvx = jnp.sum(jnp.where(sel, vxj, 0.0), axis=1, keepdims=True)
        nvy = jnp.sum(jnp.where(sel, vyj, 0.0), axis=1, keepdims=True)
        d2 = jnp.where(sel, jnp.inf, d2)
        e = ((nx - pxi) * embWT[0:1, :] + (ny - pyi) * embWT[1:2, :]
             + (nvx - vxi) * embWT[2:3, :] + (nvy - vyi) * embWT[3:4, :] + embb[...])
        xs.append(jnp.maximum(e, 0.0))
    x = jnp.concatenate(xs, axis=1)
    gates = jnp.dot(x, WihT[...], preferred_element_type=jnp.float32) + bsum[...]
    ig = jax.nn.sigmoid(gates[:, 0:_H])
    gg = jnp.tanh(gates[:, _H:2 * _H])
    og = jax.nn.sigmoid(gates[:, 2 * _H:3 * _H])
    h1 = og * jnp.tanh(ig * gg)
    out_ref[...] = jnp.dot(h1, oWT[...], preferred_element_type=jnp.float32) + ob[...]


def kernel(_hidden, obs1, obs2, emb_W, emb_b, W_ih, W_hh, b_ih, b_hh, out_W, out_b):
    del _hidden, W_hh
    o1T = obs1.T
    o2T = obs2.T
    embWT = emb_W.T
    embb = emb_b.reshape(1, -1)
    keep = jnp.concatenate([W_ih[0:_H], W_ih[2 * _H:4 * _H]], axis=0)
    WihT = keep.T
    b = (b_ih + b_hh).reshape(1, -1)
    bsum = jnp.concatenate([b[:, 0:_H], b[:, 2 * _H:4 * _H]], axis=1)
    oWT = out_W.T
    ob = out_b.reshape(1, -1)

    grid = (_N // _R,)
    full = lambda shape: pl.BlockSpec(shape, lambda i: (0, 0))
    return pl.pallas_call(
        _body,
        grid=grid,
        in_specs=[
            full((2, _N)),
            full((2, _N)),
            pl.BlockSpec((_R, 2), lambda i: (i, 0)),
            pl.BlockSpec((_R, 2), lambda i: (i, 0)),
            full((4, _OUT // _K)),
            full((1, _OUT // _K)),
            full((_OUT, 3 * _H)),
            full((1, 3 * _H)),
            full((_H, _OUT)),
            full((1, _OUT)),
        ],
        out_specs=pl.BlockSpec((_R, _OUT), lambda i: (i, 0)),
        out_shape=jax.ShapeDtypeStruct((_N, _OUT), jnp.float32),
    )(o1T, o2T, obs1, obs2, embWT, embb, WihT, bsum, oWT, ob)


# speedup vs baseline: 16.8385x; 1.4732x over previous
import jax
import jax.numpy as jnp
from jax.experimental import pallas as pl

_N = 2048
_R = 256
_H = 1024
_OUT = 256
_K = 8


def _body(o1T, o2T, o1b, o2b, pieces, embWT, embb, WihT, bsum, oWT, ob, out_ref):
    i = pl.program_id(0)
    pxj = o2T[0:1, :]
    pyj = o2T[1:2, :]
    vxj = pxj - o1T[0:1, :]
    vyj = pyj - o1T[1:2, :]
    pxi = o2b[:, 0:1]
    pyi = o2b[:, 1:2]
    vxi = pxi - o1b[:, 0:1]
    vyi = pyi - o1b[:, 1:2]
    dx = pxj - pxi
    dy = pyj - pyi
    d2 = dx * dx + dy * dy
    colio = jax.lax.broadcasted_iota(jnp.int32, (_R, _N), 1)
    rowg = i * _R + jax.lax.broadcasted_iota(jnp.int32, (_R, _N), 0)
    d2 = jnp.where(colio == rowg, jnp.inf, d2)
    xs = []
    for _ in range(_K):
        m = jnp.min(d2, axis=1, keepdims=True)
        idx = jnp.min(jnp.where(d2 == m, colio, _N), axis=1, keepdims=True)
        sel = colio == idx
        onehot = jnp.where(sel, 1.0, 0.0).astype(jnp.bfloat16)
        d2 = jnp.where(sel, jnp.inf, d2)
        p = jnp.dot(onehot, pieces[...], preferred_element_type=jnp.float32)
        g = p[:, 0:4] + p[:, 4:8] + p[:, 8:12]
        e = ((g[:, 0:1] - pxi) * embWT[0:1, :] + (g[:, 1:2] - pyi) * embWT[1:2, :]
             + (g[:, 2:3] - vxi) * embWT[2:3, :] + (g[:, 3:4] - vyi) * embWT[3:4, :]
             + embb[...])
        xs.append(jnp.maximum(e, 0.0))
    x = jnp.concatenate(xs, axis=1)
    gates = jnp.dot(x, WihT[...], preferred_element_type=jnp.float32) + bsum[...]
    ig = jax.nn.sigmoid(gates[:, 0:_H])
    gg = jnp.tanh(gates[:, _H:2 * _H])
    og = jax.nn.sigmoid(gates[:, 2 * _H:3 * _H])
    h1 = og * jnp.tanh(ig * gg)
    out_ref[...] = jnp.dot(h1, oWT[...], preferred_element_type=jnp.float32) + ob[...]


def kernel(_hidden, obs1, obs2, emb_W, emb_b, W_ih, W_hh, b_ih, b_hh, out_W, out_b):
    del _hidden, W_hh
    o1T = obs1.T
    o2T = obs2.T
    embWT = emb_W.T
    embb = emb_b.reshape(1, -1)
    keep = jnp.concatenate([W_ih[0:_H], W_ih[2 * _H:4 * _H]], axis=0)
    WihT = keep.T
    b = (b_ih + b_hh).reshape(1, -1)
    bsum = jnp.concatenate([b[:, 0:_H], b[:, 2 * _H:4 * _H]], axis=1)
    oWT = out_W.T
    ob = out_b.reshape(1, -1)
    vel = obs2 - obs1
    coords = jnp.concatenate([obs2, vel], axis=1)
    hi = coords.astype(jnp.bfloat16)
    r1 = coords - hi.astype(jnp.float32)
    mid = r1.astype(jnp.bfloat16)
    lo = (r1 - mid.astype(jnp.float32)).astype(jnp.bfloat16)
    pieces = jnp.concatenate([hi, mid, lo], axis=1)

    grid = (_N // _R,)
    full = lambda shape: pl.BlockSpec(shape, lambda i: (0, 0))
    return pl.pallas_call(
        _body,
        grid=grid,
        in_specs=[
            full((2, _N)),
            full((2, _N)),
            pl.BlockSpec((_R, 2), lambda i: (i, 0)),
            pl.BlockSpec((_R, 2), lambda i: (i, 0)),
            full((_N, 12)),
            full((4, _OUT // _K)),
            full((1, _OUT // _K)),
            full((_OUT, 3 * _H)),
            full((1, 3 * _H)),
            full((_H, _OUT)),
            full((1, _OUT)),
        ],
        out_specs=pl.BlockSpec((_R, _OUT), lambda i: (i, 0)),
        out_shape=jax.ShapeDtypeStruct((_N, _OUT), jnp.float32),
    )(o1T, o2T, obs1, obs2, pieces, embWT, embb, WihT, bsum, oWT, ob)


# speedup vs baseline: 18.4316x; 1.0946x over previous
import jax
import jax.numpy as jnp
from jax.experimental import pallas as pl

_N = 2048
_R = 512
_H = 1024
_OUT = 256
_K = 8


def _body(o1T, o2T, o1b, o2b, pieces, embWT, embb, WihT, bsum, oWT, ob, out_ref):
    i = pl.program_id(0)
    pxj = o2T[0:1, :]
    pyj = o2T[1:2, :]
    vxj = pxj - o1T[0:1, :]
    vyj = pyj - o1T[1:2, :]
    pxi = o2b[:, 0:1]
    pyi = o2b[:, 1:2]
    vxi = pxi - o1b[:, 0:1]
    vyi = pyi - o1b[:, 1:2]
    dx = pxj - pxi
    dy = pyj - pyi
    d2 = dx * dx + dy * dy
    colio = jax.lax.broadcasted_iota(jnp.int32, (_R, _N), 1)
    rowg = i * _R + jax.lax.broadcasted_iota(jnp.int32, (_R, _N), 0)
    d2 = jnp.where(colio == rowg, jnp.inf, d2)
    xs = []
    for k in range(_K):
        m = jnp.min(d2, axis=1, keepdims=True)
        idx = jnp.min(jnp.where(d2 == m, colio, _N), axis=1, keepdims=True)
        sel = colio == idx
        onehot = jnp.where(sel, 1.0, 0.0).astype(jnp.bfloat16)
        if k + 1 < _K:
            d2 = jnp.where(sel, jnp.inf, d2)
        p = jnp.dot(onehot, pieces[...], preferred_element_type=jnp.float32)
        g = p[:, 0:4] + p[:, 4:8] + p[:, 8:12]
        e = ((g[:, 0:1] - pxi) * embWT[0:1, :] + (g[:, 1:2] - pyi) * embWT[1:2, :]
             + (g[:, 2:3] - vxi) * embWT[2:3, :] + (g[:, 3:4] - vyi) * embWT[3:4, :]
             + embb[...])
        xs.append(jnp.maximum(e, 0.0))
    x = jnp.concatenate(xs, axis=1)
    gates = jnp.dot(x, WihT[...], preferred_element_type=jnp.float32) + bsum[...]
    ig = jax.nn.sigmoid(gates[:, 0:_H])
    gg = jnp.tanh(gates[:, _H:2 * _H])
    og = jax.nn.sigmoid(gates[:, 2 * _H:3 * _H])
    h1 = og * jnp.tanh(ig * gg)
    out_ref[...] = jnp.dot(h1, oWT[...], preferred_element_type=jnp.float32) + ob[...]


def kernel(_hidden, obs1, obs2, emb_W, emb_b, W_ih, W_hh, b_ih, b_hh, out_W, out_b):
    del _hidden, W_hh
    o1T = obs1.T
    o2T = obs2.T
    embWT = emb_W.T
    embb = emb_b.reshape(1, -1)
    keep = jnp.concatenate([W_ih[0:_H], W_ih[2 * _H:4 * _H]], axis=0)
    WihT = keep.T
    b = (b_ih + b_hh).reshape(1, -1)
    bsum = jnp.concatenate([b[:, 0:_H], b[:, 2 * _H:4 * _H]], axis=1)
    oWT = out_W.T
    ob = out_b.reshape(1, -1)
    vel = obs2 - obs1
    coords = jnp.concatenate([obs2, vel], axis=1)
    hi = coords.astype(jnp.bfloat16)
    r1 = coords - hi.astype(jnp.float32)
    mid = r1.astype(jnp.bfloat16)
    lo = (r1 - mid.astype(jnp.float32)).astype(jnp.bfloat16)
    pieces = jnp.concatenate([hi, mid, lo], axis=1)

    grid = (_N // _R,)
    full = lambda shape: pl.BlockSpec(shape, lambda i: (0, 0))
    return pl.pallas_call(
        _body,
        grid=grid,
        in_specs=[
            full((2, _N)),
            full((2, _N)),
            pl.BlockSpec((_R, 2), lambda i: (i, 0)),
            pl.BlockSpec((_R, 2), lambda i: (i, 0)),
            full((_N, 12)),
            full((4, _OUT // _K)),
            full((1, _OUT // _K)),
            full((_OUT, 3 * _H)),
            full((1, 3 * _H)),
            full((_H, _OUT)),
            full((1, _OUT)),
        ],
        out_specs=pl.BlockSpec((_R, _OUT), lambda i: (i, 0)),
        out_shape=jax.ShapeDtypeStruct((_N, _OUT), jnp.float32),
    )(o1T, o2T, obs1, obs2, pieces, embWT, embb, WihT, bsum, oWT, ob)


# speedup vs baseline: 19.3008x; 1.0472x over previous
import jax
import jax.numpy as jnp
from jax.experimental import pallas as pl

_N = 2048
_R = 512
_H = 1024
_OUT = 256
_K = 8


def _nt(a, b):
    return jax.lax.dot_general(a, b, (((1,), (1,)), ((), ())),
                               preferred_element_type=jnp.float32)


def _body(o1T, o2T, o1b, o2b, pieces, embWT, embb, Wih, bfull, oW, ob, out_ref):
    i = pl.program_id(0)
    pxj = o2T[0:1, :]
    pyj = o2T[1:2, :]
    vxj = pxj - o1T[0:1, :]
    vyj = pyj - o1T[1:2, :]
    pxi = o2b[:, 0:1]
    pyi = o2b[:, 1:2]
    vxi = pxi - o1b[:, 0:1]
    vyi = pyi - o1b[:, 1:2]
    dx = pxj - pxi
    dy = pyj - pyi
    d2 = dx * dx + dy * dy
    colio = jax.lax.broadcasted_iota(jnp.int32, (_R, _N), 1)
    rowg = i * _R + jax.lax.broadcasted_iota(jnp.int32, (_R, _N), 0)
    d2 = jnp.where(colio == rowg, jnp.inf, d2)
    xs = []
    for k in range(_K):
        m = jnp.min(d2, axis=1, keepdims=True)
        idx = jnp.min(jnp.where(d2 == m, colio, _N), axis=1, keepdims=True)
        sel = colio == idx
        onehot = jnp.where(sel, 1.0, 0.0).astype(jnp.bfloat16)
        if k + 1 < _K:
            d2 = jnp.where(sel, jnp.inf, d2)
        p = jnp.dot(onehot, pieces[...], preferred_element_type=jnp.float32)
        g = p[:, 0:4] + p[:, 4:8] + p[:, 8:12]
        e = ((g[:, 0:1] - pxi) * embWT[0:1, :] + (g[:, 1:2] - pyi) * embWT[1:2, :]
             + (g[:, 2:3] - vxi) * embWT[2:3, :] + (g[:, 3:4] - vyi) * embWT[3:4, :]
             + embb[...])
        xs.append(jnp.maximum(e, 0.0))
    x = jnp.concatenate(xs, axis=1)
    ig = jax.nn.sigmoid(_nt(x, Wih[0:_H, :]) + bfull[:, 0:_H])
    gg = jnp.tanh(_nt(x, Wih[2 * _H:3 * _H, :]) + bfull[:, 2 * _H:3 * _H])
    og = jax.nn.sigmoid(_nt(x, Wih[3 * _H:4 * _H, :]) + bfull[:, 3 * _H:4 * _H])
    h1 = og * jnp.tanh(ig * gg)
    out_ref[...] = _nt(h1, oW[...]) + ob[...]


def kernel(_hidden, obs1, obs2, emb_W, emb_b, W_ih, W_hh, b_ih, b_hh, out_W, out_b):
    del _hidden, W_hh
    o1T = obs1.T
    o2T = obs2.T
    embWT = emb_W.T
    embb = emb_b.reshape(1, -1)
    bfull = (b_ih + b_hh).reshape(1, -1)
    ob = out_b.reshape(1, -1)
    vel = obs2 - obs1
    coords = jnp.concatenate([obs2, vel], axis=1)
    hi = coords.astype(jnp.bfloat16)
    r1 = coords - hi.astype(jnp.float32)
    mid = r1.astype(jnp.bfloat16)
    lo = (r1 - mid.astype(jnp.float32)).astype(jnp.bfloat16)
    pieces = jnp.concatenate([hi, mid, lo], axis=1)

    grid = (_N // _R,)
    full = lambda shape: pl.BlockSpec(shape, lambda i: (0, 0))
    return pl.pallas_call(
        _body,
        grid=grid,
        in_specs=[
            full((2, _N)),
            full((2, _N)),
            pl.BlockSpec((_R, 2), lambda i: (i, 0)),
            pl.BlockSpec((_R, 2), lambda i: (i, 0)),
            full((_N, 12)),
            full((4, _OUT // _K)),
            full((1, _OUT // _K)),
            full((4 * _H, _OUT)),
            full((1, 4 * _H)),
            full((_OUT, _H)),
            full((1, _OUT)),
        ],
        out_specs=pl.BlockSpec((_R, _OUT), lambda i: (i, 0)),
        out_shape=jax.ShapeDtypeStruct((_N, _OUT), jnp.float32),
    )(o1T, o2T, obs1, obs2, pieces, embWT, embb, W_ih, bfull, out_W, ob)
